# baseline (speedup 1.0000x reference)
import jax
import jax.numpy as jnp
from functools import partial
from jax.experimental import pallas as pl
from jax.experimental.pallas import tpu as pltpu

B = 32
N = 16
EPS = 1e-5
INV_SQRT2 = 0.7071067811865476
F32 = jnp.float32
BF16 = jnp.bfloat16
G = 128
_dot = partial(jnp.dot, preferred_element_type=jnp.float32)
_dot_hi = partial(jnp.dot, preferred_element_type=jnp.float32,
                  precision=jax.lax.Precision.HIGHEST)


def _stats(P, guard, S, C):
    def body(bi, acc):
        s1, s2 = acc
        v = P[bi, :, pl.ds(guard, S)].astype(F32)
        return (s1 + jnp.sum(v, axis=1, keepdims=True),
                s2 + jnp.sum(v * v, axis=1, keepdims=True))
    z = jnp.zeros((C, 1), F32)
    s1, s2 = jax.lax.fori_loop(0, B, body, (z, z))
    denom = 1.0 / (B * S)
    mu = s1 * denom
    var = s2 * denom - mu * mu
    return mu, var


def _herpn_inplace(P, guard, S, C, hw_ref):
    mu, var = _stats(P, guard, S, C)
    s = jax.lax.rsqrt(var + EPS)
    w0 = hw_ref[0, :, 0:1]
    w1 = hw_ref[0, :, 1:2]
    w2 = hw_ref[0, :, 2:3]
    c2 = w2 * INV_SQRT2 * s * s
    c1 = w1 * s - 2.0 * mu * c2
    c0 = w0 - w2 * INV_SQRT2 + mu * mu * c2 - w1 * s * mu

    def body(bi, _):
        v = P[bi, :, pl.ds(guard, S)].astype(F32)
        r = c0 + v * (c1 + v * c2)
        P[bi, :, pl.ds(guard, S)] = r.astype(P.dtype)
        return 0
    jax.lax.fori_loop(0, B, body, 0)


def _conv_pass(Pin, Pout, wv, S, W, Cin, gout, selv=None):
    pos = jax.lax.broadcasted_iota(jnp.int32, (1, S), 1)
    wp = pos % W
    m_dx0 = (wp != 0).astype(BF16)
    m_dx2 = (wp != W - 1).astype(BF16)
    S_out = S if selv is None else 128 * selv.shape[0]

    def body(bi, _):
        taps = []
        for dy in range(3):
            for dx in range(3):
                o = W * (dy - 1) + (dx - 1)
                t = Pin[bi, :, pl.ds(G + o, S)].astype(BF16)
                if dx == 0:
                    t = t * m_dx0
                elif dx == 2:
                    t = t * m_dx2
                taps.append(t)
        rhs = jnp.concatenate(taps, axis=0)
        out = _dot(wv, rhs)
        if selv is not None:
            out = jnp.concatenate(
                [_dot_hi(out[:, 512 * j:512 * (j + 1)], selv[j])
                 for j in range(selv.shape[0])], axis=1)
        Pout[bi, :, pl.ds(gout, S_out)] = out.astype(Pout.dtype)
        return 0
    jax.lax.fori_loop(0, B, body, 0)


def _backbone_kernel(xref, w0r, h1r, w1r, h2r, w2r, h3r, w3r, h4r, w4r,
                     h5r, w5r, phr, gr, br, sel1r, sel2r, pmr, yref,
                     P0, P1, P2, P3, P4, P5, P6):
    for P, C, S in ((P0, 3, 4096), (P1, 16, 4096), (P2, 16, 4096),
                    (P3, 32, 1024), (P4, 32, 1024), (P5, 64, 256)):
        P[:, :, 0:G] = jnp.zeros((B, C, G), P.dtype)
        P[:, :, G + S:] = jnp.zeros((B, C, P.shape[2] - G - S), P.dtype)

    P0[:, :, G:G + 4096] = xref[0]

    sel1 = sel1r[...]
    sel2 = sel2r[...]

    _conv_pass(P0, P1, w0r[0], 4096, 64, 3, G)
    _herpn_inplace(P1, G, 4096, 16, h1r)
    _conv_pass(P1, P2, w1r[0], 4096, 64, 16, G)
    _herpn_inplace(P2, G, 4096, 16, h2r)
    _conv_pass(P2, P3, w2r[0], 4096, 64, 16, G, sel1)
    _herpn_inplace(P3, G, 1024, 32, h3r)
    _conv_pass(P3, P4, w3r[0], 1024, 32, 32, G)
    _herpn_inplace(P4, G, 1024, 32, h4r)
    _conv_pass(P4, P5, w4r[0], 1024, 32, 32, G, sel2)
    _herpn_inplace(P5, G, 256, 64, h5r)
    _conv_pass(P5, P6, w5r[0], 256, 16, 64, 0)
    _herpn_inplace(P6, 0, 256, 64, phr)

    v = P6[...]
    pooled = _dot_hi(v.reshape(B * 64, 256), pmr[...])
    p3 = pooled.reshape(B, 64, 4)
    mu = jnp.mean(p3, axis=0, keepdims=True)
    var = jnp.mean(p3 * p3, axis=0, keepdims=True) - mu * mu
    yref[0] = (p3 - mu) * jax.lax.rsqrt(var + EPS) * gr[0] + br[0]


def _head_kernel(y_ref, lw_ref, lb_ref, jw_ref, jb_ref, og_ref, pred_ref):
    y = y_ref[...]
    yt = y.transpose(1, 0, 2)
    flat = yt.reshape(B, N * 256)
    og = _dot(flat, lw_ref[...]) + lb_ref[0]
    mu = jnp.mean(og, axis=0)
    var = jnp.mean(og * og, axis=0) - mu * mu
    og_ref[...] = (og - mu) * jax.lax.rsqrt(var + EPS)
    rows = yt.reshape(B * N, 256)
    pred_ref[...] = (_dot(rows, jw_ref[...])
                     + jb_ref[0])


def _conv_mat(w):
    n, co, ci, _, _ = w.shape
    return w.transpose(0, 1, 3, 4, 2).reshape(n, co, 9 * ci)


def _downsel(s_in, w_in):
    w_out = w_in // 2
    s_out = s_in // 4
    ar = jnp.arange(s_out)
    src = (ar // w_out) * 2 * w_in + (ar % w_out) * 2
    m = jax.nn.one_hot(src % 512, 512, dtype=F32)
    return m.reshape(s_out // 128, 128, 512).transpose(0, 2, 1)


@jax.jit
def kernel(x, conv0_w, hw1, cw1, hw2, cw2, hw3, cw3, hw4, cw4, hw5, cw5,
           pool_hw, bn1_g, bn1_b, lin_w, lin_b, jig_w, jig_b):
    xp = (x.reshape(B, 3, 4, 64, 4, 64).transpose(2, 4, 0, 1, 3, 5)
          .reshape(N, B, 3, 4096))
    ws = [_conv_mat(w).astype(BF16) for w in (conv0_w, cw1, cw2, cw3, cw4, cw5)]
    hts = [h.transpose(0, 2, 1) for h in (hw1, hw2, hw3, hw4, hw5, pool_hw)]
    g3 = bn1_g.reshape(N, 64, 4)
    b3 = bn1_b.reshape(N, 64, 4)
    sel1 = _downsel(4096, 64)
    sel2 = _downsel(1024, 32)
    ar = jnp.arange(256)
    q = (ar // 16 // 8) * 2 + (ar % 16 // 8)
    pmat = jax.nn.one_hot(q, 4, dtype=F32) / 64.0

    def wspec(shape):
        nd = len(shape) - 1
        return pl.BlockSpec((1,) + shape[1:], lambda n: (n,) + (0,) * nd)

    def cspec(shape):
        nd = len(shape)
        return pl.BlockSpec(shape, lambda n: (0,) * nd)

    y4 = pl.pallas_call(
        _backbone_kernel,
        out_shape=jax.ShapeDtypeStruct((N, B, 64, 4), F32),
        grid=(N,),
        in_specs=[
            wspec((N, B, 3, 4096)),
            wspec(ws[0].shape), wspec(hts[0].shape),
            wspec(ws[1].shape), wspec(hts[1].shape),
            wspec(ws[2].shape), wspec(hts[2].shape),
            wspec(ws[3].shape), wspec(hts[3].shape),
            wspec(ws[4].shape), wspec(hts[4].shape),
            wspec(ws[5].shape),
            wspec(hts[5].shape), wspec(g3.shape), wspec(b3.shape),
            cspec(sel1.shape), cspec(sel2.shape), cspec(pmat.shape),
        ],
        out_specs=pl.BlockSpec((1, B, 64, 4), lambda n: (n, 0, 0, 0)),
        scratch_shapes=[
            pltpu.VMEM((B, 3, 4096 + 2 * G), F32),
            pltpu.VMEM((B, 16, 4096 + 2 * G), F32),
            pltpu.VMEM((B, 16, 4096 + 2 * G), F32),
            pltpu.VMEM((B, 32, 1024 + 2 * G), F32),
            pltpu.VMEM((B, 32, 1024 + 2 * G), F32),
            pltpu.VMEM((B, 64, 256 + 2 * G), F32),
            pltpu.VMEM((B, 64, 256), F32),
        ],
        compiler_params=pltpu.CompilerParams(
            dimension_semantics=("parallel",),
            vmem_limit_bytes=56 * 1024 * 1024,
        ),
        name="patch_backbones",
    )(xp, ws[0], hts[0], ws[1], hts[1], ws[2], hts[2], ws[3], hts[3],
      ws[4], hts[4], ws[5], hts[5], g3, b3, sel1, sel2, pmat)
    y = y4.reshape(N, B, 256)

    og, pred = pl.pallas_call(
        _head_kernel,
        out_shape=(jax.ShapeDtypeStruct((B, 256), F32),
                   jax.ShapeDtypeStruct((B * N, N), F32)),
        in_specs=[
            pl.BlockSpec((N, B, 256), lambda: (0, 0, 0)),
            pl.BlockSpec((N * 256, 256), lambda: (0, 0)),
            pl.BlockSpec((1, 256), lambda: (0, 0)),
            pl.BlockSpec((256, N), lambda: (0, 0)),
            pl.BlockSpec((1, N), lambda: (0, 0)),
        ],
        out_specs=(pl.BlockSpec((B, 256), lambda: (0, 0)),
                   pl.BlockSpec((B * N, N), lambda: (0, 0))),
        compiler_params=pltpu.CompilerParams(
            vmem_limit_bytes=48 * 1024 * 1024,
        ),
        name="patch_head",
    )(y, lin_w.T, lin_b.reshape(1, 256), jig_w.T, jig_b.reshape(1, N))

    target = jnp.tile(jnp.arange(N), B)
    return og, pred, target


# speedup vs baseline: 1.3030x; 1.3030x over previous
import jax
import jax.numpy as jnp
from functools import partial
from jax.sharding import Mesh, PartitionSpec
from jax.experimental import pallas as pl
from jax.experimental.pallas import tpu as pltpu

B = 32
N = 16
EPS = 1e-5
INV_SQRT2 = 0.7071067811865476
F32 = jnp.float32
BF16 = jnp.bfloat16
G = 128
_dot = partial(jnp.dot, preferred_element_type=jnp.float32)
_dot_hi = partial(jnp.dot, preferred_element_type=jnp.float32,
                  precision=jax.lax.Precision.HIGHEST)


def _stats(P, guard, S, C):
    def body(bi, acc):
        s1, s2 = acc
        v = P[bi, :, pl.ds(guard, S)].astype(F32)
        return (s1 + jnp.sum(v, axis=1, keepdims=True),
                s2 + jnp.sum(v * v, axis=1, keepdims=True))
    z = jnp.zeros((C, 1), F32)
    s1, s2 = jax.lax.fori_loop(0, B, body, (z, z))
    denom = 1.0 / (B * S)
    mu = s1 * denom
    var = s2 * denom - mu * mu
    return mu, var


def _herpn_inplace(P, guard, S, C, hw_ref):
    mu, var = _stats(P, guard, S, C)
    s = jax.lax.rsqrt(var + EPS)
    w0 = hw_ref[0, :, 0:1]
    w1 = hw_ref[0, :, 1:2]
    w2 = hw_ref[0, :, 2:3]
    c2 = w2 * INV_SQRT2 * s * s
    c1 = w1 * s - 2.0 * mu * c2
    c0 = w0 - w2 * INV_SQRT2 + mu * mu * c2 - w1 * s * mu

    def body(bi, _):
        v = P[bi, :, pl.ds(guard, S)].astype(F32)
        r = c0 + v * (c1 + v * c2)
        P[bi, :, pl.ds(guard, S)] = r.astype(P.dtype)
        return 0
    jax.lax.fori_loop(0, B, body, 0)


def _conv_pass(Pin, Pout, wv, S, W, Cin, gout, selv=None):
    pos = jax.lax.broadcasted_iota(jnp.int32, (1, S), 1)
    wp = pos % W
    m_dx0 = (wp != 0).astype(Pin.dtype)
    m_dx2 = (wp != W - 1).astype(Pin.dtype)
    S_out = S if selv is None else 128 * selv.shape[0]

    def body(bi, _):
        taps = []
        for dy in range(3):
            for dx in range(3):
                o = W * (dy - 1) + (dx - 1)
                t = Pin[bi, :, pl.ds(G + o, S)]
                if dx == 0:
                    t = t * m_dx0
                elif dx == 2:
                    t = t * m_dx2
                taps.append(t)
        rhs = jnp.concatenate(taps, axis=0)
        out = _dot(wv, rhs)
        if selv is not None:
            out = jnp.concatenate(
                [_dot_hi(out[:, 512 * j:512 * (j + 1)], selv[j])
                 for j in range(selv.shape[0])], axis=1)
        Pout[bi, :, pl.ds(gout, S_out)] = out.astype(Pout.dtype)
        return 0
    jax.lax.fori_loop(0, B, body, 0)


def _backbone_kernel(xref, w0r, h1r, w1r, h2r, w2r, h3r, w3r, h4r, w4r,
                     h5r, w5r, phr, gr, br, sel1r, sel2r, pmr, yref,
                     P0, P1, P2, P3, P4, P5, P6):
    for P, C, S in ((P0, 3, 4096), (P1, 16, 4096), (P2, 16, 4096),
                    (P3, 32, 1024), (P4, 32, 1024), (P5, 64, 256)):
        P[:, :, 0:G] = jnp.zeros((B, C, G), P.dtype)
        P[:, :, G + S:] = jnp.zeros((B, C, P.shape[2] - G - S), P.dtype)

    P0[:, :, G:G + 4096] = xref[0]

    sel1 = sel1r[...]
    sel2 = sel2r[...]

    _conv_pass(P0, P1, w0r[0], 4096, 64, 3, G)
    _herpn_inplace(P1, G, 4096, 16, h1r)
    _conv_pass(P1, P2, w1r[0], 4096, 64, 16, G)
    _herpn_inplace(P2, G, 4096, 16, h2r)
    _conv_pass(P2, P3, w2r[0], 4096, 64, 16, G, sel1)
    _herpn_inplace(P3, G, 1024, 32, h3r)
    _conv_pass(P3, P4, w3r[0], 1024, 32, 32, G)
    _herpn_inplace(P4, G, 1024, 32, h4r)
    _conv_pass(P4, P5, w4r[0], 1024, 32, 32, G, sel2)
    _herpn_inplace(P5, G, 256, 64, h5r)
    _conv_pass(P5, P6, w5r[0], 256, 16, 64, 0)
    _herpn_inplace(P6, 0, 256, 64, phr)

    v = P6[...]
    pooled = _dot_hi(v.reshape(B * 64, 256), pmr[...])
    p3 = pooled.reshape(B, 64, 4)
    mu = jnp.mean(p3, axis=0, keepdims=True)
    var = jnp.mean(p3 * p3, axis=0, keepdims=True) - mu * mu
    yref[0] = (p3 - mu) * jax.lax.rsqrt(var + EPS) * gr[0] + br[0]


def _head_kernel(y_ref, lw_ref, lb_ref, jw_ref, jb_ref, og_ref, pred_ref):
    y = y_ref[...]
    yt = y.transpose(1, 0, 2)
    flat = yt.reshape(B, N * 256)
    og = _dot(flat, lw_ref[...]) + lb_ref[0]
    mu = jnp.mean(og, axis=0)
    var = jnp.mean(og * og, axis=0) - mu * mu
    og_ref[...] = (og - mu) * jax.lax.rsqrt(var + EPS)
    rows = yt.reshape(B * N, 256)
    pred_ref[...] = (_dot(rows, jw_ref[...])
                     + jb_ref[0])


def _conv_mat(w):
    n, co, ci, _, _ = w.shape
    return w.transpose(0, 1, 3, 4, 2).reshape(n, co, 9 * ci)


def _downsel(s_in, w_in):
    w_out = w_in // 2
    s_out = s_in // 4
    ar = jnp.arange(s_out)
    src = (ar // w_out) * 2 * w_in + (ar % w_out) * 2
    m = jax.nn.one_hot(src % 512, 512, dtype=F32)
    return m.reshape(s_out // 128, 128, 512).transpose(0, 2, 1)


@jax.jit
def kernel(x, conv0_w, hw1, cw1, hw2, cw2, hw3, cw3, hw4, cw4, hw5, cw5,
           pool_hw, bn1_g, bn1_b, lin_w, lin_b, jig_w, jig_b):
    xp = (x.reshape(B, 3, 4, 64, 4, 64).transpose(2, 4, 0, 1, 3, 5)
          .reshape(N, B, 3, 4096))
    ws = [_conv_mat(w) for w in (conv0_w, cw1, cw2, cw3, cw4, cw5)]
    hts = [h.transpose(0, 2, 1) for h in (hw1, hw2, hw3, hw4, hw5, pool_hw)]
    g3 = bn1_g.reshape(N, 64, 4)
    b3 = bn1_b.reshape(N, 64, 4)
    sel1 = _downsel(4096, 64)
    sel2 = _downsel(1024, 32)
    ar = jnp.arange(256)
    q = (ar // 16 // 8) * 2 + (ar % 16 // 8)
    pmat = jax.nn.one_hot(q, 4, dtype=F32) / 64.0

    def _bb_call(xp, w0, h1, w1, h2, w2, h3, w3, h4, w4, h5, w5, ph,
                 g3, b3, sel1, sel2, pmat):
        nl = xp.shape[0]

        def wspec(shape):
            nd = len(shape) - 1
            return pl.BlockSpec((1,) + shape[1:], lambda n: (n,) + (0,) * nd)

        def cspec(shape):
            nd = len(shape)
            return pl.BlockSpec(shape, lambda n: (0,) * nd)

        return pl.pallas_call(
            _backbone_kernel,
            out_shape=jax.ShapeDtypeStruct((nl, B, 64, 4), F32),
            grid=(nl,),
            in_specs=[
                wspec(xp.shape),
                wspec(w0.shape), wspec(h1.shape),
                wspec(w1.shape), wspec(h2.shape),
                wspec(w2.shape), wspec(h3.shape),
                wspec(w3.shape), wspec(h4.shape),
                wspec(w4.shape), wspec(h5.shape),
                wspec(w5.shape),
                wspec(ph.shape), wspec(g3.shape), wspec(b3.shape),
                cspec(sel1.shape), cspec(sel2.shape), cspec(pmat.shape),
            ],
            out_specs=pl.BlockSpec((1, B, 64, 4), lambda n: (n, 0, 0, 0)),
            scratch_shapes=[
                pltpu.VMEM((B, 3, 4096 + 2 * G), F32),
                pltpu.VMEM((B, 16, 4096 + 2 * G), F32),
                pltpu.VMEM((B, 16, 4096 + 2 * G), F32),
                pltpu.VMEM((B, 32, 1024 + 2 * G), F32),
                pltpu.VMEM((B, 32, 1024 + 2 * G), F32),
                pltpu.VMEM((B, 64, 256 + 2 * G), F32),
                pltpu.VMEM((B, 64, 256), F32),
            ],
            compiler_params=pltpu.CompilerParams(
                dimension_semantics=("parallel",),
                vmem_limit_bytes=56 * 1024 * 1024,
            ),
            name="patch_backbones",
        )(xp, w0, h1, w1, h2, w2, h3, w3, h4, w4, h5, w5, ph, g3, b3,
          sel1, sel2, pmat)

    bb_args = (xp, ws[0], hts[0], ws[1], hts[1], ws[2], hts[2], ws[3],
               hts[3], ws[4], hts[4], ws[5], hts[5], g3, b3)
    consts = (sel1, sel2, pmat)
    devs = jax.devices()
    nd = 2 if len(devs) >= 2 else 1
    if nd > 1:
        mesh = Mesh(devs[:nd], ('d',))
        sh = PartitionSpec('d')
        rep = PartitionSpec()
        y4 = jax.shard_map(
            _bb_call, mesh=mesh,
            in_specs=(sh,) * len(bb_args) + (rep,) * len(consts),
            out_specs=sh,
            check_vma=False,
        )(*bb_args, *consts)
        y4 = jax.lax.with_sharding_constraint(
            y4, jax.sharding.NamedSharding(mesh, PartitionSpec()))
    else:
        y4 = _bb_call(*bb_args, *consts)
    y = y4.reshape(N, B, 256)

    def _head_call(y, lwt, lb2, jwt, jb2):
        return pl.pallas_call(
            _head_kernel,
            out_shape=(jax.ShapeDtypeStruct((B, 256), F32),
                       jax.ShapeDtypeStruct((B * N, N), F32)),
            in_specs=[
                pl.BlockSpec((N, B, 256), lambda: (0, 0, 0)),
                pl.BlockSpec((N * 256, 256), lambda: (0, 0)),
                pl.BlockSpec((1, 256), lambda: (0, 0)),
                pl.BlockSpec((256, N), lambda: (0, 0)),
                pl.BlockSpec((1, N), lambda: (0, 0)),
            ],
            out_specs=(pl.BlockSpec((B, 256), lambda: (0, 0)),
                       pl.BlockSpec((B * N, N), lambda: (0, 0))),
            compiler_params=pltpu.CompilerParams(
                vmem_limit_bytes=48 * 1024 * 1024,
            ),
            name="patch_head",
        )(y, lwt, lb2, jwt, jb2)

    head_args = (y, lin_w.T, lin_b.reshape(1, 256), jig_w.T,
                 jig_b.reshape(1, N))
    if nd > 1:
        rep = PartitionSpec()
        og, pred = jax.shard_map(
            _head_call, mesh=mesh,
            in_specs=(rep,) * 5, out_specs=(rep, rep),
            check_vma=False,
        )(*head_args)
    else:
        og, pred = _head_call(*head_args)

    target = jnp.tile(jnp.arange(N), B)
    return og, pred, target


# speedup vs baseline: 1.3206x; 1.0135x over previous
import jax
import jax.numpy as jnp
from functools import partial
from jax.sharding import Mesh, PartitionSpec
from jax.experimental import pallas as pl
from jax.experimental.pallas import tpu as pltpu

B = 32
N = 16
EPS = 1e-5
INV_SQRT2 = 0.7071067811865476
F32 = jnp.float32
BF16 = jnp.bfloat16
G = 128
_dot = partial(jnp.dot, preferred_element_type=jnp.float32)
_dot_hi = partial(jnp.dot, preferred_element_type=jnp.float32,
                  precision=jax.lax.Precision.HIGHEST)


def _stats(P, guard, S, C):
    def body(bi, acc):
        s1, s2 = acc
        v = P[bi, :, pl.ds(guard, S)].astype(F32)
        return (s1 + jnp.sum(v, axis=1, keepdims=True),
                s2 + jnp.sum(v * v, axis=1, keepdims=True))
    z = jnp.zeros((C, 1), F32)
    s1, s2 = jax.lax.fori_loop(0, B, body, (z, z))
    denom = 1.0 / (B * S)
    mu = s1 * denom
    var = s2 * denom - mu * mu
    return mu, var


def _herpn_inplace(P, guard, S, C, hw_ref):
    mu, var = _stats(P, guard, S, C)
    s = jax.lax.rsqrt(var + EPS)
    w0 = hw_ref[0, :, 0:1]
    w1 = hw_ref[0, :, 1:2]
    w2 = hw_ref[0, :, 2:3]
    c2 = w2 * INV_SQRT2 * s * s
    c1 = w1 * s - 2.0 * mu * c2
    c0 = w0 - w2 * INV_SQRT2 + mu * mu * c2 - w1 * s * mu

    def body(bi, _):
        v = P[bi, :, pl.ds(guard, S)].astype(F32)
        r = c0 + v * (c1 + v * c2)
        P[bi, :, pl.ds(guard, S)] = r.astype(P.dtype)
        return 0
    jax.lax.fori_loop(0, B, body, 0)


def _conv_pass(Pin, Pout, wv, S, W, Cin, gout, selv=None):
    pos = jax.lax.broadcasted_iota(jnp.int32, (1, S), 1)
    wp = pos % W
    m_dx0 = (wp != 0).astype(Pin.dtype)
    m_dx2 = (wp != W - 1).astype(Pin.dtype)
    S_out = S if selv is None else 128 * selv.shape[0]

    def body(bi, _):
        taps = []
        for dy in range(3):
            for dx in range(3):
                o = W * (dy - 1) + (dx - 1)
                t = Pin[bi, :, pl.ds(G + o, S)]
                if dx == 0:
                    t = t * m_dx0
                elif dx == 2:
                    t = t * m_dx2
                taps.append(t)
        rhs = jnp.concatenate(taps, axis=0)
        out = _dot(wv, rhs)
        if selv is not None:
            out = jnp.concatenate(
                [_dot_hi(out[:, 512 * j:512 * (j + 1)], selv[j])
                 for j in range(selv.shape[0])], axis=1)
        Pout[bi, :, pl.ds(gout, S_out)] = out.astype(Pout.dtype)
        return 0
    jax.lax.fori_loop(0, B, body, 0)


def _backbone_kernel(xref, w0r, h1r, w1r, h2r, w2r, h3r, w3r, h4r, w4r,
                     h5r, w5r, phr, gr, br, sel1r, sel2r, pmr, yref,
                     P0, P1, P2, P3, P4, P5, P6):
    for P, C, S in ((P0, 3, 4096), (P1, 16, 4096), (P2, 16, 4096),
                    (P3, 32, 1024), (P4, 32, 1024), (P5, 64, 256)):
        P[:, :, 0:G] = jnp.zeros((B, C, G), P.dtype)
        P[:, :, G + S:] = jnp.zeros((B, C, P.shape[2] - G - S), P.dtype)

    P0[:, :, G:G + 4096] = xref[0]

    sel1 = sel1r[...]
    sel2 = sel2r[...]

    _conv_pass(P0, P1, w0r[0], 4096, 64, 3, G)
    _herpn_inplace(P1, G, 4096, 16, h1r)
    _conv_pass(P1, P2, w1r[0], 4096, 64, 16, G)
    _herpn_inplace(P2, G, 4096, 16, h2r)
    _conv_pass(P2, P3, w2r[0], 4096, 64, 16, G, sel1)
    _herpn_inplace(P3, G, 1024, 32, h3r)
    _conv_pass(P3, P4, w3r[0], 1024, 32, 32, G)
    _herpn_inplace(P4, G, 1024, 32, h4r)
    _conv_pass(P4, P5, w4r[0], 1024, 32, 32, G, sel2)
    _herpn_inplace(P5, G, 256, 64, h5r)
    _conv_pass(P5, P6, w5r[0], 256, 16, 64, 0)
    _herpn_inplace(P6, 0, 256, 64, phr)

    v = P6[...]
    pooled = _dot_hi(v.reshape(B * 64, 256), pmr[...])
    p3 = pooled.reshape(B, 64, 4)
    mu = jnp.mean(p3, axis=0, keepdims=True)
    var = jnp.mean(p3 * p3, axis=0, keepdims=True) - mu * mu
    yref[0] = (p3 - mu) * jax.lax.rsqrt(var + EPS) * gr[0] + br[0]


def _head_kernel(y_ref, lw_ref, lb_ref, jw_ref, jb_ref, og_ref, pred_ref):
    y = y_ref[...]
    yt = y.transpose(1, 0, 2)
    flat = yt.reshape(B, N * 256)
    og = _dot(flat, lw_ref[...]) + lb_ref[0]
    mu = jnp.mean(og, axis=0)
    var = jnp.mean(og * og, axis=0) - mu * mu
    og_ref[...] = (og - mu) * jax.lax.rsqrt(var + EPS)
    rows = yt.reshape(B * N, 256)
    pred_ref[...] = (_dot(rows, jw_ref[...])
                     + jb_ref[0])


def _conv_mat(w):
    n, co, ci, _, _ = w.shape
    return w.transpose(0, 1, 3, 4, 2).reshape(n, co, 9 * ci)


def _downsel(s_in, w_in):
    w_out = w_in // 2
    s_out = s_in // 4
    ar = jnp.arange(s_out)
    src = (ar // w_out) * 2 * w_in + (ar % w_out) * 2
    m = jax.nn.one_hot(src % 512, 512, dtype=F32)
    return m.reshape(s_out // 128, 128, 512).transpose(0, 2, 1)


@jax.jit
def kernel(x, conv0_w, hw1, cw1, hw2, cw2, hw3, cw3, hw4, cw4, hw5, cw5,
           pool_hw, bn1_g, bn1_b, lin_w, lin_b, jig_w, jig_b):

    ws = [_conv_mat(w) for w in (conv0_w, cw1, cw2, cw3, cw4, cw5)]
    hts = [h.transpose(0, 2, 1) for h in (hw1, hw2, hw3, hw4, hw5, pool_hw)]
    g3 = bn1_g.reshape(N, 64, 4)
    b3 = bn1_b.reshape(N, 64, 4)
    sel1 = _downsel(4096, 64)
    sel2 = _downsel(1024, 32)
    ar = jnp.arange(256)
    q = (ar // 16 // 8) * 2 + (ar % 16 // 8)
    pmat = jax.nn.one_hot(q, 4, dtype=F32) / 64.0

    def _bb_call(x, w0, h1, w1, h2, w2, h3, w3, h4, w4, h5, w5, ph,
                 g3, b3, sel1, sel2, pmat):
        xp = (x.reshape(B, 3, -1, 64, 4, 64).transpose(2, 4, 0, 1, 3, 5)
              .reshape(-1, B, 3, 4096))
        nl = xp.shape[0]

        def wspec(shape):
            nd = len(shape) - 1
            return pl.BlockSpec((1,) + shape[1:], lambda n: (n,) + (0,) * nd)

        def cspec(shape):
            nd = len(shape)
            return pl.BlockSpec(shape, lambda n: (0,) * nd)

        return pl.pallas_call(
            _backbone_kernel,
            out_shape=jax.ShapeDtypeStruct((nl, B, 64, 4), F32),
            grid=(nl,),
            in_specs=[
                wspec(xp.shape),
                wspec(w0.shape), wspec(h1.shape),
                wspec(w1.shape), wspec(h2.shape),
                wspec(w2.shape), wspec(h3.shape),
                wspec(w3.shape), wspec(h4.shape),
                wspec(w4.shape), wspec(h5.shape),
                wspec(w5.shape),
                wspec(ph.shape), wspec(g3.shape), wspec(b3.shape),
                cspec(sel1.shape), cspec(sel2.shape), cspec(pmat.shape),
            ],
            out_specs=pl.BlockSpec((1, B, 64, 4), lambda n: (n, 0, 0, 0)),
            scratch_shapes=[
                pltpu.VMEM((B, 3, 4096 + 2 * G), F32),
                pltpu.VMEM((B, 16, 4096 + 2 * G), F32),
                pltpu.VMEM((B, 16, 4096 + 2 * G), F32),
                pltpu.VMEM((B, 32, 1024 + 2 * G), F32),
                pltpu.VMEM((B, 32, 1024 + 2 * G), F32),
                pltpu.VMEM((B, 64, 256 + 2 * G), F32),
                pltpu.VMEM((B, 64, 256), F32),
            ],
            compiler_params=pltpu.CompilerParams(
                dimension_semantics=("parallel",),
                vmem_limit_bytes=56 * 1024 * 1024,
            ),
            name="patch_backbones",
        )(xp, w0, h1, w1, h2, w2, h3, w3, h4, w4, h5, w5, ph, g3, b3,
          sel1, sel2, pmat)

    bb_args = (x, ws[0], hts[0], ws[1], hts[1], ws[2], hts[2], ws[3],
               hts[3], ws[4], hts[4], ws[5], hts[5], g3, b3)
    consts = (sel1, sel2, pmat)
    devs = jax.devices()
    nd = 2 if len(devs) >= 2 else 1
    if nd > 1:
        mesh = Mesh(devs[:nd], ('d',))
        sh = PartitionSpec('d')
        rep = PartitionSpec()
        y4 = jax.shard_map(
            _bb_call, mesh=mesh,
            in_specs=(PartitionSpec(None, None, 'd', None),)
                     + (sh,) * (len(bb_args) - 1) + (rep,) * len(consts),
            out_specs=sh,
            check_vma=False,
        )(*bb_args, *consts)
        y4 = jax.lax.with_sharding_constraint(
            y4, jax.sharding.NamedSharding(mesh, PartitionSpec()))
    else:
        y4 = _bb_call(*bb_args, *consts)
    y = y4.reshape(N, B, 256)

    def _head_call(y, lwt, lb2, jwt, jb2):
        return pl.pallas_call(
            _head_kernel,
            out_shape=(jax.ShapeDtypeStruct((B, 256), F32),
                       jax.ShapeDtypeStruct((B * N, N), F32)),
            in_specs=[
                pl.BlockSpec((N, B, 256), lambda: (0, 0, 0)),
                pl.BlockSpec((N * 256, 256), lambda: (0, 0)),
                pl.BlockSpec((1, 256), lambda: (0, 0)),
                pl.BlockSpec((256, N), lambda: (0, 0)),
                pl.BlockSpec((1, N), lambda: (0, 0)),
            ],
            out_specs=(pl.BlockSpec((B, 256), lambda: (0, 0)),
                       pl.BlockSpec((B * N, N), lambda: (0, 0))),
            compiler_params=pltpu.CompilerParams(
                vmem_limit_bytes=48 * 1024 * 1024,
            ),
            name="patch_head",
        )(y, lwt, lb2, jwt, jb2)

    head_args = (y, lin_w.T, lin_b.reshape(1, 256), jig_w.T,
                 jig_b.reshape(1, N))
    if nd > 1:
        rep = PartitionSpec()
        og, pred = jax.shard_map(
            _head_call, mesh=mesh,
            in_specs=(rep,) * 5, out_specs=(rep, rep),
            check_vma=False,
        )(*head_args)
    else:
        og, pred = _head_call(*head_args)

    target = jnp.tile(jnp.arange(N), B)
    return og, pred, target


# speedup vs baseline: 1.4430x; 1.0927x over previous
import jax
import jax.numpy as jnp
from functools import partial
from jax.sharding import Mesh, PartitionSpec
from jax.experimental import pallas as pl
from jax.experimental.pallas import tpu as pltpu

B = 32
N = 16
EPS = 1e-5
INV_SQRT2 = 0.7071067811865476
F32 = jnp.float32
BF16 = jnp.bfloat16
G = 128
_dot = partial(jnp.dot, preferred_element_type=jnp.float32)
_dot_hi = partial(jnp.dot, preferred_element_type=jnp.float32,
                  precision=jax.lax.Precision.HIGHEST)


def _stats(P, guard, S, C):
    def body(bi, acc):
        s1, s2 = acc
        v = P[bi, :, pl.ds(guard, S)].astype(F32)
        return (s1 + jnp.sum(v, axis=1, keepdims=True),
                s2 + jnp.sum(v * v, axis=1, keepdims=True))
    z = jnp.zeros((C, 1), F32)
    s1, s2 = jax.lax.fori_loop(0, B, body, (z, z))
    denom = 1.0 / (B * S)
    mu = s1 * denom
    var = s2 * denom - mu * mu
    return mu, var


def _herpn_inplace(P, guard, S, C, hw_ref):
    mu, var = _stats(P, guard, S, C)
    s = jax.lax.rsqrt(var + EPS)
    w0 = hw_ref[0, :, 0:1]
    w1 = hw_ref[0, :, 1:2]
    w2 = hw_ref[0, :, 2:3]
    c2 = w2 * INV_SQRT2 * s * s
    c1 = w1 * s - 2.0 * mu * c2
    c0 = w0 - w2 * INV_SQRT2 + mu * mu * c2 - w1 * s * mu

    def body(bi, _):
        v = P[bi, :, pl.ds(guard, S)].astype(F32)
        r = c0 + v * (c1 + v * c2)
        P[bi, :, pl.ds(guard, S)] = r.astype(P.dtype)
        return 0
    jax.lax.fori_loop(0, B, body, 0)


def _conv_pass(Pin, Pout, wv, S, W, Cin, gout, selv=None):
    pos = jax.lax.broadcasted_iota(jnp.int32, (1, S), 1)
    wp = pos % W
    m_dx0 = (wp != 0).astype(Pin.dtype)
    m_dx2 = (wp != W - 1).astype(Pin.dtype)
    S_out = S if selv is None else 128 * selv.shape[0]

    def body(bi, _):
        taps = []
        for dy in range(3):
            for dx in range(3):
                o = W * (dy - 1) + (dx - 1)
                t = Pin[bi, :, pl.ds(G + o, S)]
                if dx == 0:
                    t = t * m_dx0
                elif dx == 2:
                    t = t * m_dx2
                taps.append(t)
        rhs = jnp.concatenate(taps, axis=0)
        out = _dot(wv, rhs)
        if selv is not None:
            h0 = out.astype(BF16)
            r1 = out - h0.astype(F32)
            h1 = r1.astype(BF16)
            h2 = (r1 - h1.astype(F32)).astype(BF16)
            cols = []
            for j in range(selv.shape[0]):
                sl = slice(512 * j, 512 * (j + 1))
                cols.append(_dot(h0[:, sl], selv[j])
                            + _dot(h1[:, sl], selv[j])
                            + _dot(h2[:, sl], selv[j]))
            out = jnp.concatenate(cols, axis=1)
        Pout[bi, :, pl.ds(gout, S_out)] = out.astype(Pout.dtype)
        return 0
    jax.lax.fori_loop(0, B, body, 0)


def _backbone_kernel(xref, w0r, h1r, w1r, h2r, w2r, h3r, w3r, h4r, w4r,
                     h5r, w5r, phr, gr, br, sel1r, sel2r, pmr, yref,
                     P0, P1, P2, P3, P4, P5, P6):
    for P, C, S in ((P0, 3, 4096), (P1, 16, 4096), (P2, 16, 4096),
                    (P3, 32, 1024), (P4, 32, 1024), (P5, 64, 256)):
        P[:, :, 0:G] = jnp.zeros((B, C, G), P.dtype)
        P[:, :, G + S:] = jnp.zeros((B, C, P.shape[2] - G - S), P.dtype)

    P0[:, :, G:G + 4096] = xref[0]

    sel1 = sel1r[...]
    sel2 = sel2r[...]

    _conv_pass(P0, P1, w0r[0], 4096, 64, 3, G)
    _herpn_inplace(P1, G, 4096, 16, h1r)
    _conv_pass(P1, P2, w1r[0], 4096, 64, 16, G)
    _herpn_inplace(P2, G, 4096, 16, h2r)
    _conv_pass(P2, P3, w2r[0], 4096, 64, 16, G, sel1)
    _herpn_inplace(P3, G, 1024, 32, h3r)
    _conv_pass(P3, P4, w3r[0], 1024, 32, 32, G)
    _herpn_inplace(P4, G, 1024, 32, h4r)
    _conv_pass(P4, P5, w4r[0], 1024, 32, 32, G, sel2)
    _herpn_inplace(P5, G, 256, 64, h5r)
    _conv_pass(P5, P6, w5r[0], 256, 16, 64, 0)
    _herpn_inplace(P6, 0, 256, 64, phr)

    v = P6[...]
    pooled = _dot_hi(v.reshape(B * 64, 256), pmr[...])
    p3 = pooled.reshape(B, 64, 4)
    mu = jnp.mean(p3, axis=0, keepdims=True)
    var = jnp.mean(p3 * p3, axis=0, keepdims=True) - mu * mu
    yref[0] = (p3 - mu) * jax.lax.rsqrt(var + EPS) * gr[0] + br[0]


def _head_kernel(y_ref, lw_ref, lb_ref, jw_ref, jb_ref, og_ref, pred_ref):
    y = y_ref[...]
    yt = y.transpose(1, 0, 2)
    flat = yt.reshape(B, N * 256)
    og = _dot(flat, lw_ref[...]) + lb_ref[0]
    mu = jnp.mean(og, axis=0)
    var = jnp.mean(og * og, axis=0) - mu * mu
    og_ref[...] = (og - mu) * jax.lax.rsqrt(var + EPS)
    rows = yt.reshape(B * N, 256)
    pred_ref[...] = (_dot(rows, jw_ref[...])
                     + jb_ref[0])


def _conv_mat(w):
    n, co, ci, _, _ = w.shape
    return w.transpose(0, 1, 3, 4, 2).reshape(n, co, 9 * ci)


def _downsel(s_in, w_in):
    w_out = w_in // 2
    s_out = s_in // 4
    ar = jnp.arange(s_out)
    src = (ar // w_out) * 2 * w_in + (ar % w_out) * 2
    m = jax.nn.one_hot(src % 512, 512, dtype=F32)
    return m.reshape(s_out // 128, 128, 512).transpose(0, 2, 1).astype(BF16)


@jax.jit
def kernel(x, conv0_w, hw1, cw1, hw2, cw2, hw3, cw3, hw4, cw4, hw5, cw5,
           pool_hw, bn1_g, bn1_b, lin_w, lin_b, jig_w, jig_b):

    ws = [_conv_mat(w) for w in (conv0_w, cw1, cw2, cw3, cw4, cw5)]
    hts = [h.transpose(0, 2, 1) for h in (hw1, hw2, hw3, hw4, hw5, pool_hw)]
    g3 = bn1_g.reshape(N, 64, 4)
    b3 = bn1_b.reshape(N, 64, 4)
    sel1 = _downsel(4096, 64)
    sel2 = _downsel(1024, 32)
    ar = jnp.arange(256)
    q = (ar // 16 // 8) * 2 + (ar % 16 // 8)
    pmat = jax.nn.one_hot(q, 4, dtype=F32) / 64.0

    def _bb_call(x, w0, h1, w1, h2, w2, h3, w3, h4, w4, h5, w5, ph,
                 g3, b3, sel1, sel2, pmat):
        xp = (x.reshape(B, 3, -1, 64, 4, 64).transpose(2, 4, 0, 1, 3, 5)
              .reshape(-1, B, 3, 4096))
        nl = xp.shape[0]

        def wspec(shape):
            nd = len(shape) - 1
            return pl.BlockSpec((1,) + shape[1:], lambda n: (n,) + (0,) * nd)

        def cspec(shape):
            nd = len(shape)
            return pl.BlockSpec(shape, lambda n: (0,) * nd)

        return pl.pallas_call(
            _backbone_kernel,
            out_shape=jax.ShapeDtypeStruct((nl, B, 64, 4), F32),
            grid=(nl,),
            in_specs=[
                wspec(xp.shape),
                wspec(w0.shape), wspec(h1.shape),
                wspec(w1.shape), wspec(h2.shape),
                wspec(w2.shape), wspec(h3.shape),
                wspec(w3.shape), wspec(h4.shape),
                wspec(w4.shape), wspec(h5.shape),
                wspec(w5.shape),
                wspec(ph.shape), wspec(g3.shape), wspec(b3.shape),
                cspec(sel1.shape), cspec(sel2.shape), cspec(pmat.shape),
            ],
            out_specs=pl.BlockSpec((1, B, 64, 4), lambda n: (n, 0, 0, 0)),
            scratch_shapes=[
                pltpu.VMEM((B, 3, 4096 + 2 * G), F32),
                pltpu.VMEM((B, 16, 4096 + 2 * G), F32),
                pltpu.VMEM((B, 16, 4096 + 2 * G), F32),
                pltpu.VMEM((B, 32, 1024 + 2 * G), F32),
                pltpu.VMEM((B, 32, 1024 + 2 * G), F32),
                pltpu.VMEM((B, 64, 256 + 2 * G), F32),
                pltpu.VMEM((B, 64, 256), F32),
            ],
            compiler_params=pltpu.CompilerParams(
                dimension_semantics=("parallel",),
                vmem_limit_bytes=56 * 1024 * 1024,
            ),
            name="patch_backbones",
        )(xp, w0, h1, w1, h2, w2, h3, w3, h4, w4, h5, w5, ph, g3, b3,
          sel1, sel2, pmat)

    bb_args = (x, ws[0], hts[0], ws[1], hts[1], ws[2], hts[2], ws[3],
               hts[3], ws[4], hts[4], ws[5], hts[5], g3, b3)
    consts = (sel1, sel2, pmat)
    devs = jax.devices()
    nd = 2 if len(devs) >= 2 else 1
    if nd > 1:
        mesh = Mesh(devs[:nd], ('d',))
        sh = PartitionSpec('d')
        rep = PartitionSpec()
        y4 = jax.shard_map(
            _bb_call, mesh=mesh,
            in_specs=(PartitionSpec(None, None, 'd', None),)
                     + (sh,) * (len(bb_args) - 1) + (rep,) * len(consts),
            out_specs=sh,
            check_vma=False,
        )(*bb_args, *consts)
        y4 = jax.lax.with_sharding_constraint(
            y4, jax.sharding.NamedSharding(mesh, PartitionSpec()))
    else:
        y4 = _bb_call(*bb_args, *consts)
    y = y4.reshape(N, B, 256)

    def _head_call(y, lwt, lb2, jwt, jb2):
        return pl.pallas_call(
            _head_kernel,
            out_shape=(jax.ShapeDtypeStruct((B, 256), F32),
                       jax.ShapeDtypeStruct((B * N, N), F32)),
            in_specs=[
                pl.BlockSpec((N, B, 256), lambda: (0, 0, 0)),
                pl.BlockSpec((N * 256, 256), lambda: (0, 0)),
                pl.BlockSpec((1, 256), lambda: (0, 0)),
                pl.BlockSpec((256, N), lambda: (0, 0)),
                pl.BlockSpec((1, N), lambda: (0, 0)),
            ],
            out_specs=(pl.BlockSpec((B, 256), lambda: (0, 0)),
                       pl.BlockSpec((B * N, N), lambda: (0, 0))),
            compiler_params=pltpu.CompilerParams(
                vmem_limit_bytes=48 * 1024 * 1024,
            ),
            name="patch_head",
        )(y, lwt, lb2, jwt, jb2)

    head_args = (y, lin_w.T, lin_b.reshape(1, 256), jig_w.T,
                 jig_b.reshape(1, N))
    if nd > 1:
        rep = PartitionSpec()
        og, pred = jax.shard_map(
            _head_call, mesh=mesh,
            in_specs=(rep,) * 5, out_specs=(rep, rep),
            check_vma=False,
        )(*head_args)
    else:
        og, pred = _head_call(*head_args)

    target = jnp.tile(jnp.arange(N), B)
    return og, pred, target
